# baseline (speedup 1.0000x reference)
import jax, jax.numpy as jnp
from functools import partial
_es = partial(jnp.einsum, precision=jax.lax.Precision.HIGHEST)

B, M, D, E, LAM, N_ITERS, TOL = 2, 50000, 8, 800000, 1.0, 6, 1e-6

def _batch_dot(a, b):
    return jnp.sum(a * b, axis=(1, 2), keepdims=True)

def kernel(c0, src, dst, R_src, R_dst):
    c0_f = c0.astype(jnp.float32)
    C = _es('eab,ead->ebd', R_src, R_dst)
    As = _es('eab,ead->ebd', R_src, R_src)
    Ad = _es('eab,ead->ebd', R_dst, R_dst)
    Dg = jnp.zeros((M, D, D), jnp.float32).at[src].add(As).at[dst].add(Ad)

    def matvec(p):
        dense = _es('iad,bid->bia', Dg, p)
        pd = p[:, dst, :]; ps = p[:, src, :]
        u = _es('ead,bed->bea', C, pd)
        v = _es('ead,bea->bed', C, ps)
        S = jnp.zeros_like(p).at[:, src, :].add(u).at[:, dst, :].add(v)
        return p + LAM * (dense - S)

    x = c0_f
    r = x - matvec(x)
    p = r
    rsold = _batch_dot(r, r)
    done = jnp.zeros((), dtype=jnp.bool_)
    for _ in range(N_ITERS):
        Ap = matvec(p)
        denom = _batch_dot(p, Ap) + 1e-12
        alpha = rsold / denom
        x_new = x + alpha * p
        r_new = r - alpha * Ap
        rsnew = _batch_dot(r_new, r_new)
        hit = jnp.sqrt(jnp.mean(rsnew)) < TOL
        p_new = r_new + rsnew / (rsold + 1e-12) * p
        x = jnp.where(done, x, x_new)
        r = jnp.where(done, r, r_new)
        p = jnp.where(done | hit, p, p_new)
        rsold = jnp.where(done | hit, rsold, rsnew)
        done = done | hit
    return x.astype(c0.dtype)


# speedup vs baseline: 49.5134x; 49.5134x over previous
import functools

import jax
import jax.numpy as jnp
from jax import lax
from jax.experimental import pallas as pl
from jax.experimental.pallas import tpu as pltpu
from jax.experimental.pallas import tpu_sc as plsc

B = 2
M = 50000
D = 8
DP = 16
E = 800000
LAM = 1.0
N_ITERS = 6
TOL = 1e-6

NC = 2
NS = 16
NW = NC * NS
L = 16
IB = 128

W_E = 128
NCH = 200
EW = NCH * W_E
E_PAD = NW * EW
NPAIR = W_E // 2
PAIR_F = 32 * L
CHUNK_F = NPAIR * PAIR_F

M_PAD = 50048
RPT = M_PAD // NS


def _rne16(x):
    i = lax.bitcast_convert_type(x, jnp.uint32)
    lsb = (i >> 16) & jnp.uint32(1)
    i = i + jnp.uint32(0x7FFF) + lsb
    i = i & jnp.uint32(0xFFFF0000)
    return lax.bitcast_convert_type(i, jnp.float32)


def _sc_matvec_body(ed, idxs, p0, zrows, out,
                    idxb, rps0, rpd0, edb,
                    ocs0, ocd0, acc0, sem):
    c = lax.axis_index("c")
    s = lax.axis_index("s")
    wid = s * NC + c

    zbase = s * RPT
    pltpu.sync_copy(zrows, acc0.at[pl.ds(zbase, RPT)])
    plsc.subcore_barrier()

    lanes = lax.iota(jnp.int32, L)
    lo8 = lanes < 8
    land7 = lanes & 7
    hb_idx = [jnp.where(lo8, k, 8 + k).reshape(L, 1) for k in range(D)]
    dup_lo = land7.reshape(L, 1)
    dup_hi = (8 + land7).reshape(L, 1)
    dn = lax.GatherDimensionNumbers(offset_dims=(), collapsed_slice_dims=(0,),
                                    start_index_map=(0,))

    def perm(v, idx):
        return lax.gather(v, idx, dn, (1,),
                          mode=lax.GatherScatterMode.PROMISE_IN_BOUNDS)

    def chunk(i, carry):
        off = wid * EW + i * W_E
        blk = wid * NCH + i
        pltpu.sync_copy(idxs.at[pl.ds(blk * 8, 8)], idxb)
        cps = [
            pltpu.async_copy(ed.at[pl.ds(off * 256, CHUNK_F)], edb, sem),
            pltpu.async_copy(p0.at[idxb.at[0]], rps0, sem),
            pltpu.async_copy(p0.at[idxb.at[1]], rpd0, sem),
        ]
        for cp in cps:
            cp.wait()

        def pair(q, qcarry):
            base = q * PAIR_F
            for rps, rpd, ocs, ocd in ((rps0, rpd0, ocs0, ocd0),):
                pps = jnp.where(lo8, rps[2 * q, :], rps[2 * q + 1, :])
                ppd = jnp.where(lo8, rpd[2 * q, :], rpd[2 * q + 1, :])
                ts = jnp.zeros((L,), jnp.float32)
                td = jnp.zeros((L,), jnp.float32)
                for d in range(D):
                    cs_d = edb[pl.ds(base + d * L, L)]
                    cd_d = edb[pl.ds(base + 128 + d * L, L)]
                    ts = ts + cs_d * perm(pps, hb_idx[d])
                    td = td + cd_d * perm(ppd, hb_idx[d])
                rr = _rne16(ts - td)
                cs = jnp.zeros((L,), jnp.float32)
                cd = jnp.zeros((L,), jnp.float32)
                for a in range(D):
                    h = perm(rr, hb_idx[a])
                    cs = cs + edb[pl.ds(base + 256 + a * L, L)] * h
                    cd = cd + edb[pl.ds(base + 384 + a * L, L)] * h
                ocs[2 * q, :] = perm(cs, dup_lo)
                ocs[2 * q + 1, :] = perm(cs, dup_hi)
                ocd[2 * q, :] = perm(cd, dup_lo)
                ocd[2 * q + 1, :] = perm(cd, dup_hi)
            return qcarry

        lax.fori_loop(0, NPAIR, pair, 0)

        scs = [
            pltpu.async_copy(ocs0, acc0.at[idxb.at[0]], sem, add=True),
            pltpu.async_copy(ocd0, acc0.at[idxb.at[1]], sem, add=True),
        ]
        for cp in scs:
            cp.wait()
        return carry

    lax.fori_loop(0, NCH, chunk, 0)
    plsc.subcore_barrier()

    obase = s * RPT
    pltpu.sync_copy(acc0.at[pl.ds(obase, RPT)],
                    out.at[c, pl.ds(obase, RPT)])


@functools.partial(
    pl.kernel,
    out_type=jax.ShapeDtypeStruct((NC, M_PAD, DP), jnp.float32),
    mesh=plsc.VectorSubcoreMesh(core_axis_name="c", subcore_axis_name="s"),
    compiler_params=pltpu.CompilerParams(use_tc_tiling_on_sc=False),
    scratch_types=[
        pltpu.VMEM((8, IB), jnp.int32),
        pltpu.VMEM((W_E, DP), jnp.float32),
        pltpu.VMEM((W_E, DP), jnp.float32),
        pltpu.VMEM((CHUNK_F,), jnp.float32),
        pltpu.VMEM((W_E, DP), jnp.float32),
        pltpu.VMEM((W_E, DP), jnp.float32),
        pltpu.VMEM_SHARED((M_PAD, DP), jnp.float32),
        pltpu.SemaphoreType.DMA,
    ],
)
def _sc_matvec(ed, idxs, p0, zrows, out, *scratch):
    _sc_matvec_body(ed, idxs, p0, zrows, out, *scratch)


def _batch_dot(a, b):
    return jnp.sum(a * b, axis=(1, 2), keepdims=True)


def kernel(c0, src, dst, R_src, R_dst):
    c0_f = c0.astype(jnp.float32)

    Rs = _rne16(jnp.pad(R_src, ((0, E_PAD - E), (0, 0), (0, 0))))
    Rd = _rne16(jnp.pad(R_dst, ((0, E_PAD - E), (0, 0), (0, 0))))
    Q = E_PAD // 2
    Rs_p = Rs.reshape(Q, 2, D, D)
    Rd_p = Rd.reshape(Q, 2, D, D)
    cps = Rs_p.transpose(0, 3, 1, 2)
    cpd = Rd_p.transpose(0, 3, 1, 2)
    rps = Rs_p.transpose(0, 2, 1, 3)
    rpdn = (-Rd_p).transpose(0, 2, 1, 3)
    ed = jnp.concatenate(
        [x.reshape(Q, D, L) for x in (cps, cpd, rps, rpdn)], axis=1
    ).reshape(-1)

    src_p = jnp.pad(src, (0, E_PAD - E)).reshape(NW * NCH, 1, IB)
    dst_p = jnp.pad(dst, (0, E_PAD - E)).reshape(NW * NCH, 1, IB)
    idxs = jnp.concatenate(
        [src_p, dst_p, jnp.zeros((NW * NCH, 6, IB), jnp.int32)], axis=1
    ).reshape(-1, IB)
    zrows = jnp.zeros((RPT, DP), jnp.float32)

    def matvec(p):
        pb = _rne16(p)
        pdup = jnp.concatenate([pb, pb], axis=-1)
        accs = []
        for b in range(B):
            part = _sc_matvec(ed, idxs, pdup[b], zrows)
            accs.append((part[0] + part[1])[:, :D])
        return p + LAM * jnp.stack(accs)

    x = jnp.pad(c0_f, ((0, 0), (0, M_PAD - M), (0, 0)))
    r = x - matvec(x)
    p = r
    rsold = _batch_dot(r, r)
    done = jnp.zeros((), dtype=jnp.bool_)
    for _ in range(N_ITERS):
        Ap = matvec(p)
        denom = _batch_dot(p, Ap) + 1e-12
        alpha = rsold / denom
        x_new = x + alpha * p
        r_new = r - alpha * Ap
        rsnew = _batch_dot(r_new, r_new)
        hit = jnp.sqrt(jnp.mean(rsnew)) < TOL
        p_new = r_new + rsnew / (rsold + 1e-12) * p
        x = jnp.where(done, x, x_new)
        r = jnp.where(done, r, r_new)
        p = jnp.where(done | hit, p, p_new)
        rsold = jnp.where(done | hit, rsold, rsnew)
        done = done | hit
    return x[:, :M, :].astype(c0.dtype)


# speedup vs baseline: 64.0429x; 1.2934x over previous
import functools

import jax
import jax.numpy as jnp
from jax import lax
from jax.experimental import pallas as pl
from jax.experimental.pallas import tpu as pltpu
from jax.experimental.pallas import tpu_sc as plsc

B = 2
M = 50000
D = 8
DP = 16
E = 800000
LAM = 1.0
N_ITERS = 6
TOL = 1e-6

NC = 2
NS = 16
NW = NC * NS
L = 16
IB = 128

W_E = 128
NCH = 200
EW = NCH * W_E
E_PAD = NW * EW
NPAIR = W_E // 2
PAIR_F = 32 * L
CHUNK_F = NPAIR * PAIR_F

M_PAD = 50048
RPT = M_PAD // NS


def _rne16(x):
    i = lax.bitcast_convert_type(x, jnp.uint32)
    lsb = (i >> 16) & jnp.uint32(1)
    i = i + jnp.uint32(0x7FFF) + lsb
    i = i & jnp.uint32(0xFFFF0000)
    return lax.bitcast_convert_type(i, jnp.float32)


def _sc_matvec_body(ed, idxs, p0, zrows, out,
                    idxba, rpsa, rpda, edba,
                    idxbb, rpsb, rpdb, edbb,
                    ocs0, ocd0, acc0, semla, semlb, sems):
    c = lax.axis_index("c")
    s = lax.axis_index("s")
    wid = s * NC + c

    zbase = s * RPT
    pltpu.sync_copy(zrows, acc0.at[pl.ds(zbase, RPT)])
    plsc.subcore_barrier()

    lanes = lax.iota(jnp.int32, L)
    lo8 = lanes < 8
    land7 = lanes & 7
    hb_idx = [jnp.where(lo8, k, 8 + k).reshape(L, 1) for k in range(D)]
    dup_lo = land7.reshape(L, 1)
    dup_hi = (8 + land7).reshape(L, 1)
    dn = lax.GatherDimensionNumbers(offset_dims=(), collapsed_slice_dims=(0,),
                                    start_index_map=(0,))

    def perm(v, idx):
        return lax.gather(v, idx, dn, (1,),
                          mode=lax.GatherScatterMode.PROMISE_IN_BOUNDS)

    def issue_loads(i, idxb, rps, rpd, edb, sem):
        off = wid * EW + i * W_E
        blk = wid * NCH + i
        pltpu.sync_copy(idxs.at[pl.ds(blk * 8, 8)], idxb)
        pltpu.async_copy(ed.at[pl.ds(off * 256, CHUNK_F)], edb, sem)
        pltpu.async_copy(p0.at[idxb.at[0]], rps, sem)
        pltpu.async_copy(p0.at[idxb.at[1]], rpd, sem)

    def wait_loads(rps, rpd, edb, sem):
        pltpu.make_async_copy(ed.at[pl.ds(0, CHUNK_F)], edb, sem).wait()
        pltpu.make_async_copy(p0.at[pl.ds(0, W_E)], rps, sem).wait()
        pltpu.make_async_copy(p0.at[pl.ds(0, W_E)], rpd, sem).wait()

    def process(idxb, rps, rpd, edb):

        def pair(q, qcarry):
            base = q * PAIR_F
            for ocs, ocd in ((ocs0, ocd0),):
                pps = jnp.where(lo8, rps[2 * q, :], rps[2 * q + 1, :])
                ppd = jnp.where(lo8, rpd[2 * q, :], rpd[2 * q + 1, :])
                ts = jnp.zeros((L,), jnp.float32)
                td = jnp.zeros((L,), jnp.float32)
                for d in range(D):
                    cs_d = edb[pl.ds(base + d * L, L)]
                    cd_d = edb[pl.ds(base + 128 + d * L, L)]
                    ts = ts + cs_d * perm(pps, hb_idx[d])
                    td = td + cd_d * perm(ppd, hb_idx[d])
                rr = _rne16(ts - td)
                cs = jnp.zeros((L,), jnp.float32)
                cd = jnp.zeros((L,), jnp.float32)
                for a in range(D):
                    h = perm(rr, hb_idx[a])
                    cs = cs + edb[pl.ds(base + 256 + a * L, L)] * h
                    cd = cd + edb[pl.ds(base + 384 + a * L, L)] * h
                ocs[2 * q, :] = perm(cs, dup_lo)
                ocs[2 * q + 1, :] = perm(cs, dup_hi)
                ocd[2 * q, :] = perm(cd, dup_lo)
                ocd[2 * q + 1, :] = perm(cd, dup_hi)
            return qcarry

        lax.fori_loop(0, NPAIR, pair, 0)

        scs = [
            pltpu.async_copy(ocs0, acc0.at[idxb.at[0]], sems, add=True),
            pltpu.async_copy(ocd0, acc0.at[idxb.at[1]], sems, add=True),
        ]
        for cp in scs:
            cp.wait()

    def chunk2(i, carry):
        issue_loads(2 * i + 1, idxbb, rpsb, rpdb, edbb, semlb)
        wait_loads(rpsa, rpda, edba, semla)
        process(idxba, rpsa, rpda, edba)
        issue_loads(2 * i + 2, idxba, rpsa, rpda, edba, semla)
        wait_loads(rpsb, rpdb, edbb, semlb)
        process(idxbb, rpsb, rpdb, edbb)
        return carry

    issue_loads(0, idxba, rpsa, rpda, edba, semla)
    lax.fori_loop(0, NCH // 2, chunk2, 0)
    wait_loads(rpsa, rpda, edba, semla)
    plsc.subcore_barrier()

    obase = s * RPT
    pltpu.sync_copy(acc0.at[pl.ds(obase, RPT)],
                    out.at[c, pl.ds(obase, RPT)])


@functools.partial(
    pl.kernel,
    out_type=jax.ShapeDtypeStruct((NC, M_PAD, DP), jnp.float32),
    mesh=plsc.VectorSubcoreMesh(core_axis_name="c", subcore_axis_name="s"),
    compiler_params=pltpu.CompilerParams(use_tc_tiling_on_sc=False),
    scratch_types=[
        pltpu.VMEM((8, IB), jnp.int32),
        pltpu.VMEM((W_E, DP), jnp.float32),
        pltpu.VMEM((W_E, DP), jnp.float32),
        pltpu.VMEM((CHUNK_F,), jnp.float32),
        pltpu.VMEM((8, IB), jnp.int32),
        pltpu.VMEM((W_E, DP), jnp.float32),
        pltpu.VMEM((W_E, DP), jnp.float32),
        pltpu.VMEM((CHUNK_F,), jnp.float32),
        pltpu.VMEM((W_E, DP), jnp.float32),
        pltpu.VMEM((W_E, DP), jnp.float32),
        pltpu.VMEM_SHARED((M_PAD, DP), jnp.float32),
        pltpu.SemaphoreType.DMA,
        pltpu.SemaphoreType.DMA,
        pltpu.SemaphoreType.DMA,
    ],
)
def _sc_matvec(ed, idxs, p0, zrows, out, *scratch):
    _sc_matvec_body(ed, idxs, p0, zrows, out, *scratch)


def _batch_dot(a, b):
    return jnp.sum(a * b, axis=(1, 2), keepdims=True)


def kernel(c0, src, dst, R_src, R_dst):
    c0_f = c0.astype(jnp.float32)

    Rs = _rne16(jnp.pad(R_src, ((0, E_PAD - E), (0, 0), (0, 0))))
    Rd = _rne16(jnp.pad(R_dst, ((0, E_PAD - E), (0, 0), (0, 0))))
    Q = E_PAD // 2
    Rs_p = Rs.reshape(Q, 2, D, D)
    Rd_p = Rd.reshape(Q, 2, D, D)
    cps = Rs_p.transpose(0, 3, 1, 2)
    cpd = Rd_p.transpose(0, 3, 1, 2)
    rps = Rs_p.transpose(0, 2, 1, 3)
    rpdn = (-Rd_p).transpose(0, 2, 1, 3)
    ed = jnp.concatenate(
        [x.reshape(Q, D, L) for x in (cps, cpd, rps, rpdn)], axis=1
    ).reshape(-1)
    ed = jnp.pad(ed, (0, CHUNK_F))

    src_p = jnp.pad(src, (0, E_PAD - E)).reshape(NW * NCH, 1, IB)
    dst_p = jnp.pad(dst, (0, E_PAD - E)).reshape(NW * NCH, 1, IB)
    idxs = jnp.concatenate(
        [src_p, dst_p, jnp.zeros((NW * NCH, 6, IB), jnp.int32)], axis=1
    ).reshape(-1, IB)
    idxs = jnp.pad(idxs, ((0, 8), (0, 0)))
    zrows = jnp.zeros((RPT, DP), jnp.float32)

    def matvec(p):
        pb = _rne16(p)
        pdup = jnp.concatenate([pb, pb], axis=-1)
        accs = []
        for b in range(B):
            part = _sc_matvec(ed, idxs, pdup[b], zrows)
            accs.append((part[0] + part[1])[:, :D])
        return p + LAM * jnp.stack(accs)

    x = jnp.pad(c0_f, ((0, 0), (0, M_PAD - M), (0, 0)))
    r = x - matvec(x)
    p = r
    rsold = _batch_dot(r, r)
    done = jnp.zeros((), dtype=jnp.bool_)
    for _ in range(N_ITERS):
        Ap = matvec(p)
        denom = _batch_dot(p, Ap) + 1e-12
        alpha = rsold / denom
        x_new = x + alpha * p
        r_new = r - alpha * Ap
        rsnew = _batch_dot(r_new, r_new)
        hit = jnp.sqrt(jnp.mean(rsnew)) < TOL
        p_new = r_new + rsnew / (rsold + 1e-12) * p
        x = jnp.where(done, x, x_new)
        r = jnp.where(done, r, r_new)
        p = jnp.where(done | hit, p, p_new)
        rsold = jnp.where(done | hit, rsold, rsnew)
        done = done | hit
    return x[:, :M, :].astype(c0.dtype)
